# baseline (speedup 1.0000x reference)
import functools

import jax
import jax.numpy as jnp
from jax.experimental import pallas as pl
from jax.experimental.pallas import tpu as pltpu
from jax.experimental.pallas import tpu_sc as plsc

E = 64
K = 2
D = 1024
F = 768
T = 128
A = T * K
R = 8
NT = (A - E) // R + E
PR = NT * R


def _router_kernel(x_ref, rw_ref, ct_ref, te_ref, ntot_ref, src_ref):
    x = x_ref[...]
    logits = jnp.dot(x, rw_ref[...], preferred_element_type=jnp.float32)
    probs = jax.nn.softmax(logits, axis=-1)

    col = jax.lax.broadcasted_iota(jnp.int32, (T, E), 1)
    i1 = jnp.argmax(probs, axis=1).reshape(T, 1)
    oh1 = (col == i1)
    m1 = jnp.sum(jnp.where(oh1, probs, 0.0), axis=1).reshape(T, 1)
    probs2 = jnp.where(oh1, -1.0, probs)
    i2 = jnp.argmax(probs2, axis=1).reshape(T, 1)
    oh2 = (col == i2)
    m2 = jnp.sum(jnp.where(oh2, probs2, 0.0), axis=1).reshape(T, 1)
    denom = m1 + m2
    w1 = m1 / denom
    w2 = m2 / denom

    e_a = jnp.concatenate([i1, i2], axis=0)
    w_a = jnp.concatenate([w1, w2], axis=0)

    colA = jax.lax.broadcasted_iota(jnp.int32, (A, E), 1)
    Aoh = (colA == e_a).astype(jnp.float32)

    ri = jax.lax.broadcasted_iota(jnp.int32, (A, A), 0)
    rj = jax.lax.broadcasted_iota(jnp.int32, (A, A), 1)
    L = (rj < ri).astype(jnp.float32)
    pref = jnp.dot(L, Aoh, preferred_element_type=jnp.float32)
    rank = jnp.sum(pref * Aoh, axis=1).reshape(A, 1)

    counts = jnp.sum(Aoh, axis=0).reshape(1, E)
    ntiles = jnp.floor((counts + (R - 1)) * (1.0 / R))
    ui = jax.lax.broadcasted_iota(jnp.int32, (E, E), 0)
    uj = jax.lax.broadcasted_iota(jnp.int32, (E, E), 1)
    U = (ui < uj).astype(jnp.float32)
    first_tile = jnp.dot(ntiles, U, preferred_element_type=jnp.float32)
    cum_incl = first_tile + ntiles

    ft_a = jnp.dot(Aoh, first_tile.reshape(E, 1),
                   preferred_element_type=jnp.float32)
    pos = ft_a * R + rank

    t8 = jax.lax.broadcasted_iota(jnp.int32, (E, NT), 1)
    cmp = (cum_incl.reshape(E, 1).astype(jnp.int32) <= t8).astype(jnp.int32)
    te = jnp.minimum(jnp.sum(cmp, axis=0).reshape(1, NT), E - 1)
    te_ref[...] = te
    ntot_ref[...] = cum_incl[:, E - 1:E].astype(jnp.int32)

    prow = jax.lax.broadcasted_iota(jnp.int32, (PR, A), 0)
    pos_i = pos.astype(jnp.int32)
    S = (prow == pos_i.reshape(1, A)).astype(jnp.float32)

    W2 = S * w_a.reshape(1, A)
    ct_ref[...] = W2[:, :T] + W2[:, T:]

    pcol = jax.lax.broadcasted_iota(jnp.int32, (A, PR), 1)
    St = (pcol == pos_i).astype(jnp.float32)
    acol = jax.lax.broadcasted_iota(jnp.int32, (1, A), 1)
    tokp1 = ((acol % T) + 1).astype(jnp.float32)
    srcf = jnp.dot(tokp1, St, preferred_element_type=jnp.float32)
    src_ref[...] = jnp.where(srcf == 0.0, float(T), srcf - 1.0).astype(jnp.int32)


def _moe_kernel(te_ref, ntot_ref, px_ref, g_ref, u_ref, d_ref, ct_ref, o_ref):
    t = pl.program_id(0)

    @pl.when(t == 0)
    def _init():
        o_ref[...] = jnp.zeros_like(o_ref)

    @pl.when(t < ntot_ref[0])
    def _compute():
        xt = px_ref[...]
        g = jnp.dot(xt, g_ref[0], preferred_element_type=jnp.float32)
        u = jnp.dot(xt, u_ref[0], preferred_element_type=jnp.float32)
        h = (g * jax.lax.logistic(g)) * u
        y = jnp.dot(h, d_ref[0], preferred_element_type=jnp.float32)
        ct = ct_ref[...]
        o_ref[...] += jax.lax.dot_general(
            ct, y, (((0,), (0,)), ((), ())),
            preferred_element_type=jnp.float32)


def _sc_gather(x_aug, src):
    HALF = PR // 2

    @pl.kernel(
        out_type=jax.ShapeDtypeStruct((PR, D), jnp.float32),
        mesh=plsc.ScalarSubcoreMesh(axis_name="core", num_cores=2),
        scratch_types=[
            pltpu.SMEM((PR,), jnp.int32),
            pltpu.SemaphoreType.DMA,
            pltpu.SemaphoreType.DMA,
        ],
    )
    def gather_kernel(x_hbm, i_hbm, o_hbm, idx_ref, isem, gsem):
        c = jax.lax.axis_index("core")
        pltpu.async_copy(i_hbm.at[0], idx_ref, isem).wait()
        base = c * HALF

        @pl.loop(0, HALF)
        def _issue(i):
            p = base + i
            pltpu.make_async_copy(x_hbm.at[idx_ref[p]], o_hbm.at[p],
                                  gsem).start()

        @pl.loop(0, HALF)
        def _drain(i):
            p = base + i
            pltpu.make_async_copy(x_hbm.at[idx_ref[p]], o_hbm.at[p],
                                  gsem).wait()

    return gather_kernel(x_aug, src)


@functools.partial(jax.jit, static_argnames=())
def kernel(hidden_states, router_weight, gate_proj, up_proj, down_proj):
    b, s, d = hidden_states.shape
    x = hidden_states.reshape(T, D)

    ct, te, ntot, src = pl.pallas_call(
        _router_kernel,
        out_shape=[
            jax.ShapeDtypeStruct((PR, T), jnp.float32),
            jax.ShapeDtypeStruct((1, NT), jnp.int32),
            jax.ShapeDtypeStruct((1, 1), jnp.int32),
            jax.ShapeDtypeStruct((1, PR), jnp.int32),
        ],
    )(x, router_weight)

    x_aug = jnp.concatenate([x, jnp.zeros((8, D), jnp.float32)], axis=0)
    px = _sc_gather(x_aug, src)

    out = pl.pallas_call(
        _moe_kernel,
        grid_spec=pltpu.PrefetchScalarGridSpec(
            num_scalar_prefetch=2,
            grid=(NT,),
            in_specs=[
                pl.BlockSpec((R, D), lambda t, te, nt: (t, 0)),
                pl.BlockSpec((1, D, F), lambda t, te, nt: (te[t], 0, 0)),
                pl.BlockSpec((1, D, F), lambda t, te, nt: (te[t], 0, 0)),
                pl.BlockSpec((1, F, D), lambda t, te, nt: (te[t], 0, 0)),
                pl.BlockSpec((R, T), lambda t, te, nt: (t, 0)),
            ],
            out_specs=pl.BlockSpec((T, D), lambda t, te, nt: (0, 0)),
        ),
        out_shape=jax.ShapeDtypeStruct((T, D), jnp.float32),
    )(te.reshape(NT), ntot.reshape(1), px, gate_proj, up_proj, down_proj, ct)

    return out.reshape(b, s, d)


# speedup vs baseline: 1.1681x; 1.1681x over previous
import functools

import jax
import jax.numpy as jnp
from jax.experimental import pallas as pl
from jax.experimental.pallas import tpu as pltpu
from jax.experimental.pallas import tpu_sc as plsc

E = 64
K = 2
D = 1024
F = 768
T = 128
A = T * K
R = 8
NT = (A - E) // R + E
PR = NT * R
SR = 128
SRQ = PR * (D // SR) // SR


def _router_kernel(x_ref, rw_ref, ct_ref, te_ref, ntot_ref, src_ref):
    x = x_ref[...]
    logits = jnp.dot(x, rw_ref[...], preferred_element_type=jnp.float32)
    probs = jax.nn.softmax(logits, axis=-1)

    col = jax.lax.broadcasted_iota(jnp.int32, (T, E), 1)
    i1 = jnp.argmax(probs, axis=1).reshape(T, 1)
    oh1 = (col == i1)
    m1 = jnp.sum(jnp.where(oh1, probs, 0.0), axis=1).reshape(T, 1)
    probs2 = jnp.where(oh1, -1.0, probs)
    i2 = jnp.argmax(probs2, axis=1).reshape(T, 1)
    oh2 = (col == i2)
    m2 = jnp.sum(jnp.where(oh2, probs2, 0.0), axis=1).reshape(T, 1)
    denom = m1 + m2
    w1 = m1 / denom
    w2 = m2 / denom

    e_a = jnp.concatenate([i1, i2], axis=0)
    w_a = jnp.concatenate([w1, w2], axis=0)

    colA = jax.lax.broadcasted_iota(jnp.int32, (A, E), 1)
    Aoh = (colA == e_a).astype(jnp.float32)

    ri = jax.lax.broadcasted_iota(jnp.int32, (A, A), 0)
    rj = jax.lax.broadcasted_iota(jnp.int32, (A, A), 1)
    L = (rj < ri).astype(jnp.float32)
    pref = jnp.dot(L, Aoh, preferred_element_type=jnp.float32)
    rank = jnp.sum(pref * Aoh, axis=1).reshape(A, 1)

    counts = jnp.sum(Aoh, axis=0).reshape(1, E)
    ntiles = jnp.floor((counts + (R - 1)) * (1.0 / R))
    ui = jax.lax.broadcasted_iota(jnp.int32, (E, E), 0)
    uj = jax.lax.broadcasted_iota(jnp.int32, (E, E), 1)
    U = (ui < uj).astype(jnp.float32)
    first_tile = jnp.dot(ntiles, U, preferred_element_type=jnp.float32)
    cum_incl = first_tile + ntiles

    ft_a = jnp.dot(Aoh, first_tile.reshape(E, 1),
                   preferred_element_type=jnp.float32)
    pos = ft_a * R + rank

    t8 = jax.lax.broadcasted_iota(jnp.int32, (E, NT), 1)
    cmp = (cum_incl.reshape(E, 1).astype(jnp.int32) <= t8).astype(jnp.int32)
    te = jnp.minimum(jnp.sum(cmp, axis=0).reshape(1, NT), E - 1)
    te_ref[...] = te
    ntot_ref[...] = cum_incl[:, E - 1:E].astype(jnp.int32)

    prow = jax.lax.broadcasted_iota(jnp.int32, (PR, A), 0)
    pos_i = pos.astype(jnp.int32)
    S = (prow == pos_i.reshape(1, A)).astype(jnp.float32)

    W2 = S * w_a.reshape(1, A)
    ct_ref[...] = W2[:, :T] + W2[:, T:]

    posr = pos_i.reshape(1, A)
    tokr = jax.lax.broadcasted_iota(jnp.int32, (1, A), 1) % T
    i44 = jax.lax.broadcasted_iota(jnp.int32, (SRQ, A), 0)
    M1 = jnp.where(i44 == posr // (PR // SRQ),
                   (tokr + 1).astype(jnp.float32), 0.0)
    jA = jax.lax.broadcasted_iota(jnp.int32, (A, SR), 1)
    M2 = (jA // 8 == pos_i % (PR // SRQ)).astype(jnp.float32)
    srcf8 = jnp.dot(M1, M2, preferred_element_type=jnp.float32)
    r_of = (jax.lax.broadcasted_iota(jnp.int32, (SRQ, SR), 1) % 8)
    src8 = jnp.where(srcf8 == 0.0, float(T * 8), (srcf8 - 1.0) * 8.0)
    src_ref[...] = src8.astype(jnp.int32) + r_of


def _moe_kernel(te_ref, ntot_ref, px_ref, g_ref, u_ref, d_ref, ct_ref, o_ref):
    t = pl.program_id(0)

    @pl.when(t == 0)
    def _init():
        o_ref[...] = jnp.zeros_like(o_ref)

    @pl.when(t < ntot_ref[0])
    def _compute():
        xt = px_ref[...]
        g = jnp.dot(xt, g_ref[0], preferred_element_type=jnp.float32)
        u = jnp.dot(xt, u_ref[0], preferred_element_type=jnp.float32)
        h = (g * jax.lax.logistic(g)) * u
        y = jnp.dot(h, d_ref[0], preferred_element_type=jnp.float32)
        ct = ct_ref[...]
        o_ref[...] += jax.lax.dot_general(
            ct, y, (((0,), (0,)), ((), ())),
            preferred_element_type=jnp.float32)


def _sc_gather(x_aug, src8):
    x8 = x_aug.reshape((T + 8) * (D // SR), SR)

    @pl.kernel(
        out_type=jax.ShapeDtypeStruct((PR * (D // SR), SR), jnp.float32),
        mesh=plsc.VectorSubcoreMesh(core_axis_name="core",
                                    subcore_axis_name="subcore"),
    )
    def gather_kernel(x_hbm, i_hbm, o_hbm):
        def body(i_vmem, o_vmem):
            pltpu.sync_copy(x_hbm.at[i_vmem.at[0]], o_vmem)

        pltpu.emit_pipeline(
            body,
            grid=(SRQ,),
            in_specs=[pl.BlockSpec((1, SR), index_map=lambda i: (i, 0))],
            out_specs=[pl.BlockSpec((SR, SR), index_map=lambda i: (i, 0))],
            core_axis_name="subcore",
            dimension_semantics=(pltpu.PARALLEL,),
        )(i_hbm, o_hbm)

    return gather_kernel(x8, src8).reshape(PR, D)


@functools.partial(jax.jit, static_argnames=())
def kernel(hidden_states, router_weight, gate_proj, up_proj, down_proj):
    b, s, d = hidden_states.shape
    x = hidden_states.reshape(T, D)

    ct, te, ntot, src = pl.pallas_call(
        _router_kernel,
        out_shape=[
            jax.ShapeDtypeStruct((PR, T), jnp.float32),
            jax.ShapeDtypeStruct((1, NT), jnp.int32),
            jax.ShapeDtypeStruct((1, 1), jnp.int32),
            jax.ShapeDtypeStruct((SRQ, SR), jnp.int32),
        ],
    )(x, router_weight)

    x_aug = jnp.concatenate([x, jnp.zeros((8, D), jnp.float32)], axis=0)
    px = _sc_gather(x_aug, src)

    out = pl.pallas_call(
        _moe_kernel,
        grid_spec=pltpu.PrefetchScalarGridSpec(
            num_scalar_prefetch=2,
            grid=(NT,),
            in_specs=[
                pl.BlockSpec((R, D), lambda t, te, nt: (t, 0)),
                pl.BlockSpec((1, D, F), lambda t, te, nt: (te[t], 0, 0)),
                pl.BlockSpec((1, D, F), lambda t, te, nt: (te[t], 0, 0)),
                pl.BlockSpec((1, F, D), lambda t, te, nt: (te[t], 0, 0)),
                pl.BlockSpec((R, T), lambda t, te, nt: (t, 0)),
            ],
            out_specs=pl.BlockSpec((T, D), lambda t, te, nt: (0, 0)),
        ),
        out_shape=jax.ShapeDtypeStruct((T, D), jnp.float32),
    )(te.reshape(NT), ntot.reshape(1), px, gate_proj, up_proj, down_proj, ct)

    return out.reshape(b, s, d)


# speedup vs baseline: 1.5623x; 1.3375x over previous
import functools

import jax
import jax.numpy as jnp
from jax.experimental import pallas as pl
from jax.experimental.pallas import tpu as pltpu

E = 64
K = 2
D = 1024
F = 768
T = 128
A = T * K
R = 8
NT = (A - E) // R + E
PR = NT * R


def _router_kernel(x_ref, rw_ref, px_ref, ct_ref, te_ref, ntot_ref):
    x = x_ref[...]
    logits = jnp.dot(x, rw_ref[...], preferred_element_type=jnp.float32)
    probs = jax.nn.softmax(logits, axis=-1)

    col = jax.lax.broadcasted_iota(jnp.int32, (T, E), 1)
    i1 = jnp.argmax(probs, axis=1).reshape(T, 1)
    oh1 = (col == i1)
    m1 = jnp.sum(jnp.where(oh1, probs, 0.0), axis=1).reshape(T, 1)
    probs2 = jnp.where(oh1, -1.0, probs)
    i2 = jnp.argmax(probs2, axis=1).reshape(T, 1)
    oh2 = (col == i2)
    m2 = jnp.sum(jnp.where(oh2, probs2, 0.0), axis=1).reshape(T, 1)
    denom = m1 + m2
    w1 = m1 / denom
    w2 = m2 / denom

    e_a = jnp.concatenate([i1, i2], axis=0)
    w_a = jnp.concatenate([w1, w2], axis=0)

    colA = jax.lax.broadcasted_iota(jnp.int32, (A, E), 1)
    Aoh = (colA == e_a).astype(jnp.float32)

    ri = jax.lax.broadcasted_iota(jnp.int32, (A, A), 0)
    rj = jax.lax.broadcasted_iota(jnp.int32, (A, A), 1)
    L = (rj < ri).astype(jnp.float32)
    pref = jnp.dot(L, Aoh, preferred_element_type=jnp.float32)
    rank = jnp.sum(pref * Aoh, axis=1).reshape(A, 1)

    counts = jnp.sum(Aoh, axis=0).reshape(1, E)
    ntiles = jnp.floor((counts + (R - 1)) * (1.0 / R))
    ui = jax.lax.broadcasted_iota(jnp.int32, (E, E), 0)
    uj = jax.lax.broadcasted_iota(jnp.int32, (E, E), 1)
    U = (ui < uj).astype(jnp.float32)
    first_tile = jnp.dot(ntiles, U, preferred_element_type=jnp.float32)
    cum_incl = first_tile + ntiles

    ft_a = jnp.dot(Aoh, first_tile.reshape(E, 1),
                   preferred_element_type=jnp.float32)
    pos = ft_a * R + rank

    t8 = jax.lax.broadcasted_iota(jnp.int32, (E, NT), 1)
    cmp = (cum_incl.reshape(E, 1).astype(jnp.int32) <= t8).astype(jnp.int32)
    te = jnp.minimum(jnp.sum(cmp, axis=0).reshape(1, NT), E - 1)
    te_ref[...] = te
    ntot_ref[...] = cum_incl[:, E - 1:E].astype(jnp.int32)

    prow = jax.lax.broadcasted_iota(jnp.int32, (PR, A), 0)
    pos_i = pos.astype(jnp.int32)
    S = (prow == pos_i.reshape(1, A)).astype(jnp.float32)

    x2 = jnp.concatenate([x, x], axis=0)
    px_ref[...] = jnp.dot(S, x2, preferred_element_type=jnp.float32)

    W2 = S * w_a.reshape(1, A)
    ct_ref[...] = W2[:, :T] + W2[:, T:]


def _moe_kernel(te_ref, ntot_ref, px_ref, g_ref, u_ref, d_ref, ct_ref, o_ref):
    t = pl.program_id(0)

    @pl.when(t == 0)
    def _init():
        o_ref[...] = jnp.zeros_like(o_ref)

    @pl.when(t < ntot_ref[0])
    def _compute():
        xt = px_ref[...]
        g = jnp.dot(xt, g_ref[0], preferred_element_type=jnp.float32)
        u = jnp.dot(xt, u_ref[0], preferred_element_type=jnp.float32)
        h = (g * jax.lax.logistic(g)) * u
        y = jnp.dot(h, d_ref[0], preferred_element_type=jnp.float32)
        ct = ct_ref[...]
        o_ref[...] += jax.lax.dot_general(
            ct, y, (((0,), (0,)), ((), ())),
            preferred_element_type=jnp.float32)


@functools.partial(jax.jit, static_argnames=())
def kernel(hidden_states, router_weight, gate_proj, up_proj, down_proj):
    b, s, d = hidden_states.shape
    x = hidden_states.reshape(T, D)

    px, ct, te, ntot = pl.pallas_call(
        _router_kernel,
        out_shape=[
            jax.ShapeDtypeStruct((PR, D), jnp.float32),
            jax.ShapeDtypeStruct((PR, T), jnp.float32),
            jax.ShapeDtypeStruct((1, NT), jnp.int32),
            jax.ShapeDtypeStruct((1, 1), jnp.int32),
        ],
    )(x, router_weight)

    out = pl.pallas_call(
        _moe_kernel,
        grid_spec=pltpu.PrefetchScalarGridSpec(
            num_scalar_prefetch=2,
            grid=(NT,),
            in_specs=[
                pl.BlockSpec((R, D), lambda t, te, nt: (t, 0)),
                pl.BlockSpec((1, D, F), lambda t, te, nt: (te[t], 0, 0)),
                pl.BlockSpec((1, D, F), lambda t, te, nt: (te[t], 0, 0)),
                pl.BlockSpec((1, F, D), lambda t, te, nt: (te[t], 0, 0)),
                pl.BlockSpec((R, T), lambda t, te, nt: (t, 0)),
            ],
            out_specs=pl.BlockSpec((T, D), lambda t, te, nt: (0, 0)),
        ),
        out_shape=jax.ShapeDtypeStruct((T, D), jnp.float32),
    )(te.reshape(NT), ntot.reshape(1), px, gate_proj, up_proj, down_proj, ct)

    return out.reshape(b, s, d)


# speedup vs baseline: 1.6827x; 1.0771x over previous
import functools

import jax
import jax.numpy as jnp
from jax.experimental import pallas as pl
from jax.experimental.pallas import tpu as pltpu

E = 64
K = 2
D = 1024
F = 768
T = 128
A = T * K
R = 8
NT = (A - E) // R + E
PR = NT * R


def _router_kernel(x_ref, rw_ref, px_ref, ct_ref, fr_ref, ntl_ref):
    x = x_ref[...]
    logits = jnp.dot(x, rw_ref[...], preferred_element_type=jnp.float32)
    probs = jax.nn.softmax(logits, axis=-1)

    col = jax.lax.broadcasted_iota(jnp.int32, (T, E), 1)
    i1 = jnp.argmax(probs, axis=1).reshape(T, 1)
    oh1 = (col == i1)
    m1 = jnp.sum(jnp.where(oh1, probs, 0.0), axis=1).reshape(T, 1)
    probs2 = jnp.where(oh1, -1.0, probs)
    i2 = jnp.argmax(probs2, axis=1).reshape(T, 1)
    oh2 = (col == i2)
    m2 = jnp.sum(jnp.where(oh2, probs2, 0.0), axis=1).reshape(T, 1)
    denom = m1 + m2
    w1 = m1 / denom
    w2 = m2 / denom

    e_a = jnp.concatenate([i1, i2], axis=0)
    w_a = jnp.concatenate([w1, w2], axis=0)

    colA = jax.lax.broadcasted_iota(jnp.int32, (A, E), 1)
    Aoh = (colA == e_a).astype(jnp.float32)

    ri = jax.lax.broadcasted_iota(jnp.int32, (A, A), 0)
    rj = jax.lax.broadcasted_iota(jnp.int32, (A, A), 1)
    L = (rj < ri).astype(jnp.float32)
    pref = jnp.dot(L, Aoh, preferred_element_type=jnp.float32)
    rank = jnp.sum(pref * Aoh, axis=1).reshape(A, 1)

    counts = jnp.sum(Aoh, axis=0).reshape(1, E)
    ntiles = jnp.floor((counts + (R - 1)) * (1.0 / R))
    ui = jax.lax.broadcasted_iota(jnp.int32, (E, E), 0)
    uj = jax.lax.broadcasted_iota(jnp.int32, (E, E), 1)
    U = (ui < uj).astype(jnp.float32)
    first_tile = jnp.dot(ntiles, U, preferred_element_type=jnp.float32)
    cum_incl = first_tile + ntiles

    ft_a = jnp.dot(Aoh, first_tile.reshape(E, 1),
                   preferred_element_type=jnp.float32)
    pos = ft_a * R + rank

    fr_ref[...] = (first_tile * R).astype(jnp.int32)
    ntl_ref[...] = ntiles.astype(jnp.int32)

    prow = jax.lax.broadcasted_iota(jnp.int32, (PR, A), 0)
    pos_i = pos.astype(jnp.int32)
    S = (prow == pos_i.reshape(1, A)).astype(jnp.float32)

    x2 = jnp.concatenate([x, x], axis=0)
    px_ref[...] = jnp.dot(S, x2, preferred_element_type=jnp.float32)

    W2 = S * w_a.reshape(1, A)
    ct_ref[...] = W2[:, :T] + W2[:, T:]


NBUF = 3


def _moe_ring_kernel(fr_ref, ntl_ref, px_ref, ct_ref, g_hbm, u_hbm, d_hbm,
                     o_ref, g_buf, u_buf, d_buf, g_sem, u_sem, d_sem):
    o_ref[...] = jnp.zeros_like(o_ref)

    def start(e, slot):
        @pl.when(ntl_ref[0, e] > 0)
        def _():
            pltpu.make_async_copy(g_hbm.at[e], g_buf.at[slot],
                                  g_sem.at[slot]).start()
            pltpu.make_async_copy(u_hbm.at[e], u_buf.at[slot],
                                  u_sem.at[slot]).start()
            pltpu.make_async_copy(d_hbm.at[e], d_buf.at[slot],
                                  d_sem.at[slot]).start()

    for e0 in range(NBUF):
        start(e0, e0)

    def expert_body(e, carry):
        slot = jax.lax.rem(e, NBUF)
        nt = ntl_ref[0, e]

        @pl.when(nt > 0)
        def _():
            pltpu.make_async_copy(g_hbm.at[e], g_buf.at[slot],
                                  g_sem.at[slot]).wait()
            pltpu.make_async_copy(u_hbm.at[e], u_buf.at[slot],
                                  u_sem.at[slot]).wait()
            pltpu.make_async_copy(d_hbm.at[e], d_buf.at[slot],
                                  d_sem.at[slot]).wait()
            base = fr_ref[0, e]

            def tile_body(i, c2):
                row = pl.multiple_of(base + i * R, R)
                xt = px_ref[pl.ds(row, R), :]
                ct = ct_ref[pl.ds(row, R), :]
                for sl in range(NBUF):
                    @pl.when(slot == sl)
                    def _():
                        g = jnp.dot(xt, g_buf[sl],
                                    preferred_element_type=jnp.float32)
                        u = jnp.dot(xt, u_buf[sl],
                                    preferred_element_type=jnp.float32)
                        h = (g * jax.lax.logistic(g)) * u
                        y = jnp.dot(h, d_buf[sl],
                                    preferred_element_type=jnp.float32)
                        o_ref[...] += jax.lax.dot_general(
                            ct, y, (((0,), (0,)), ((), ())),
                            preferred_element_type=jnp.float32)
                return c2

            jax.lax.fori_loop(0, nt, tile_body, 0)

        @pl.when(e + NBUF < E)
        def _():
            start(e + NBUF, slot)
        return carry

    jax.lax.fori_loop(0, E, expert_body, 0)


@functools.partial(jax.jit, static_argnames=())
def kernel(hidden_states, router_weight, gate_proj, up_proj, down_proj):
    b, s, d = hidden_states.shape
    x = hidden_states.reshape(T, D)

    px, ct, fr, ntl = pl.pallas_call(
        _router_kernel,
        out_shape=[
            jax.ShapeDtypeStruct((PR, D), jnp.float32),
            jax.ShapeDtypeStruct((PR, T), jnp.float32),
            jax.ShapeDtypeStruct((1, E), jnp.int32),
            jax.ShapeDtypeStruct((1, E), jnp.int32),
        ],
    )(x, router_weight)

    out = pl.pallas_call(
        _moe_ring_kernel,
        in_specs=[
            pl.BlockSpec(memory_space=pltpu.SMEM),
            pl.BlockSpec(memory_space=pltpu.SMEM),
            pl.BlockSpec(memory_space=pltpu.VMEM),
            pl.BlockSpec(memory_space=pltpu.VMEM),
            pl.BlockSpec(memory_space=pltpu.HBM),
            pl.BlockSpec(memory_space=pltpu.HBM),
            pl.BlockSpec(memory_space=pltpu.HBM),
        ],
        out_specs=pl.BlockSpec(memory_space=pltpu.VMEM),
        out_shape=jax.ShapeDtypeStruct((T, D), jnp.float32),
        scratch_shapes=[
            pltpu.VMEM((NBUF, D, F), jnp.float32),
            pltpu.VMEM((NBUF, D, F), jnp.float32),
            pltpu.VMEM((NBUF, F, D), jnp.float32),
            pltpu.SemaphoreType.DMA((NBUF,)),
            pltpu.SemaphoreType.DMA((NBUF,)),
            pltpu.SemaphoreType.DMA((NBUF,)),
        ],
    )(fr, ntl, px, ct, gate_proj, up_proj, down_proj)

    return out.reshape(b, s, d)
